# baseline (speedup 1.0000x reference)
import jax
import jax.numpy as jnp
from jax.experimental import pallas as pl
from jax.experimental.pallas import tpu as pltpu

_NF = 64
_ND = 100000
_B = 1024
_BN = 1000
_NB = _ND // _BN
_INV_TEMP = 20.0


def _loss_kernel(x_ref, t_ref, f_ref, out_ref, m_ref, s_ref, p_ref):
    j = pl.program_id(0)

    @pl.when(j == 0)
    def _init():
        m_ref[...] = jnp.full((_B, 1), -jnp.inf, jnp.float32)
        s_ref[...] = jnp.zeros((_B, 1), jnp.float32)
        p_ref[...] = jnp.zeros((1, 1), jnp.float32)

    x = x_ref[...]
    f = f_ref[...]
    logits = jax.lax.dot_general(
        x, f, (((1,), (1,)), ((), ())),
        preferred_element_type=jnp.float32)

    col = j * _BN + jax.lax.broadcasted_iota(jnp.int32, (_B, _BN), 1)
    t = t_ref[...]
    p_ref[...] += jnp.sum(jnp.where(col == t, logits, 0.0),
                          axis=(0, 1), keepdims=True)

    m_old = m_ref[...]
    m_new = jnp.maximum(m_old, jnp.max(logits, axis=1, keepdims=True))
    s_ref[...] = (s_ref[...] * jnp.exp(m_old - m_new)
                  + jnp.sum(jnp.exp(logits - m_new), axis=1, keepdims=True))
    m_ref[...] = m_new

    @pl.when(j == _NB - 1)
    def _fin():
        logz = m_ref[...] + jnp.log(s_ref[...])
        out_ref[...] = (jnp.sum(logz, axis=(0, 1), keepdims=True)
                        - p_ref[...]) / _B


def kernel(inputs, targets, features):
    x = inputs * _INV_TEMP
    t = targets.reshape(_B, 1)
    out = pl.pallas_call(
        _loss_kernel,
        grid=(_NB,),
        in_specs=[
            pl.BlockSpec((_B, _NF), lambda j: (0, 0)),
            pl.BlockSpec((_B, 1), lambda j: (0, 0)),
            pl.BlockSpec((_BN, _NF), lambda j: (j, 0)),
        ],
        out_specs=pl.BlockSpec((1, 1), lambda j: (0, 0)),
        out_shape=jax.ShapeDtypeStruct((1, 1), jnp.float32),
        scratch_shapes=[
            pltpu.VMEM((_B, 1), jnp.float32),
            pltpu.VMEM((_B, 1), jnp.float32),
            pltpu.VMEM((1, 1), jnp.float32),
        ],
    )(x, t, features)
    return out[0, 0]


# speedup vs baseline: 1.3168x; 1.3168x over previous
import functools

import jax
import jax.numpy as jnp
from jax import lax
from jax.experimental import pallas as pl
from jax.experimental.pallas import tpu as pltpu
from jax.experimental.pallas import tpu_sc as plsc

_NF = 64
_ND = 100000
_B = 1024
_BN = 2000
_NB = _ND // _BN
_INV_TEMP = 20.0

_SC_INFO = plsc.get_sparse_core_info()
_NW = _SC_INFO.num_cores * _SC_INFO.num_subcores
_BPW = _B // _NW
_L = _SC_INFO.num_lanes


@functools.partial(
    pl.kernel,
    mesh=plsc.VectorSubcoreMesh(core_axis_name="c", subcore_axis_name="s"),
    out_type=jax.ShapeDtypeStruct((_B, 2 * _NF), jnp.float32),
    scratch_types=[
        pltpu.VMEM((_BPW,), jnp.int32),
        pltpu.VMEM((_BPW, 2 * _NF), jnp.float32),
        pltpu.SemaphoreType.DMA,
    ],
)
def _sc_gather(t_hbm, f2_hbm, out_hbm, idx_v, rows_v, sem):
    wid = lax.axis_index("s") * _SC_INFO.num_cores + lax.axis_index("c")
    base = wid * _BPW
    pltpu.sync_copy(t_hbm.at[pl.ds(base, _BPW)], idx_v)
    for c in range(_BPW // _L):
        sl = pl.ds(c * _L, _L)
        idx_v[sl] = lax.shift_right_logical(idx_v[sl], 1)
    pltpu.async_copy(f2_hbm.at[idx_v], rows_v, sem).wait()
    pltpu.sync_copy(rows_v, out_hbm.at[pl.ds(base, _BPW)])


def _loss_kernel(x_ref, f_ref, t_ref, g_ref, out_ref, m_ref, s_ref):
    j = pl.program_id(0)

    @pl.when(j == 0)
    def _init():
        m_ref[...] = jnp.full((_B, 1), -jnp.inf, jnp.float32)
        s_ref[...] = jnp.zeros((_B, 1), jnp.float32)

    x = x_ref[...]
    f = f_ref[...]
    logits = lax.dot_general(
        x, f, (((1,), (1,)), ((), ())),
        preferred_element_type=jnp.float32)

    m_old = m_ref[...]
    m_new = jnp.maximum(m_old, jnp.max(logits, axis=1, keepdims=True))
    e = jnp.exp(logits - m_new)
    ones = jnp.ones((_BN, 1), jnp.float32)
    bsum = lax.dot_general(
        e, ones, (((1,), (0,)), ((), ())),
        preferred_element_type=jnp.float32)
    s_ref[...] = s_ref[...] * jnp.exp(m_old - m_new) + bsum
    m_ref[...] = m_new

    @pl.when(j == _NB - 1)
    def _fin():
        g2 = g_ref[...]
        odd = (t_ref[...] & 1) == 1
        grow = jnp.where(odd, g2[:, _NF:], g2[:, :_NF])
        picked = jnp.sum(x * grow, axis=1, keepdims=True)
        logz = m_ref[...] + jnp.log(s_ref[...])
        out_ref[...] = jnp.sum(logz - picked, axis=(0, 1), keepdims=True) / _B


def kernel(inputs, targets, features):
    x = inputs * _INV_TEMP
    f2 = features.reshape(_ND // 2, 2 * _NF)
    g2 = _sc_gather(targets, f2)
    t = targets.reshape(_B, 1)
    out = pl.pallas_call(
        _loss_kernel,
        grid=(_NB,),
        in_specs=[
            pl.BlockSpec((_B, _NF), lambda j: (0, 0)),
            pl.BlockSpec((_BN, _NF), lambda j: (j, 0)),
            pl.BlockSpec((_B, 1), lambda j: (0, 0)),
            pl.BlockSpec((_B, 2 * _NF), lambda j: (0, 0)),
        ],
        out_specs=pl.BlockSpec((1, 1), lambda j: (0, 0)),
        out_shape=jax.ShapeDtypeStruct((1, 1), jnp.float32),
        scratch_shapes=[
            pltpu.VMEM((_B, 1), jnp.float32),
            pltpu.VMEM((_B, 1), jnp.float32),
        ],
    )(x, features, t, g2)
    return out[0, 0]


# speedup vs baseline: 1.3437x; 1.0204x over previous
import functools

import jax
import jax.numpy as jnp
from jax import lax
from jax.experimental import pallas as pl
from jax.experimental.pallas import tpu as pltpu
from jax.experimental.pallas import tpu_sc as plsc

_NF = 64
_ND = 100000
_B = 1024
_BN = 2000
_NB = _ND // _BN
_INV_TEMP = 20.0
_LN2 = 0.6931471805599453
_LOG2E = 1.4426950408889634
_SHIFT = 100.0

_NC = 2
_NS = 16
_L = 16
_NW = _NC * _NS
_BPW = _B // _NW


@functools.partial(
    pl.kernel,
    mesh=plsc.VectorSubcoreMesh(core_axis_name="c", subcore_axis_name="s"),
    out_type=jax.ShapeDtypeStruct((_B, 2 * _NF), jnp.float32),
    scratch_types=[
        pltpu.VMEM((_BPW,), jnp.int32),
        pltpu.VMEM((_BPW, 2 * _NF), jnp.float32),
        pltpu.SemaphoreType.DMA,
    ],
)
def _sc_gather(t_hbm, f2_hbm, out_hbm, idx_v, rows_v, sem):
    wid = lax.axis_index("s") * _NC + lax.axis_index("c")
    base = wid * _BPW
    pltpu.sync_copy(t_hbm.at[pl.ds(base, _BPW)], idx_v)
    for c in range(_BPW // _L):
        sl = pl.ds(c * _L, _L)
        idx_v[sl] = lax.shift_right_logical(idx_v[sl], 1)
    pltpu.async_copy(f2_hbm.at[idx_v], rows_v, sem).wait()
    pltpu.sync_copy(rows_v, out_hbm.at[pl.ds(base, _BPW)])


def _loss_kernel(x_ref, f_ref, c_ref, t_ref, g_ref, out_ref, s_ref, buf_ref):
    j = pl.program_id(0)

    @pl.when(j == 0)
    def _init():
        s_ref[...] = jnp.zeros((_B, 1), jnp.float32)

    x = x_ref[...]

    prev = buf_ref[...]
    e = jnp.exp2(prev - c_ref[...])
    ones = jnp.ones((_BN, 1), jnp.float32)
    bsum = lax.dot_general(
        e, ones, (((1,), (0,)), ((), ())),
        preferred_element_type=jnp.float32)
    s_ref[...] += jnp.where(j > 0, bsum, 0.0)

    f = f_ref[...]
    logits = lax.dot_general(
        x.astype(jnp.bfloat16), f.astype(jnp.bfloat16),
        (((1,), (1,)), ((), ())),
        preferred_element_type=jnp.float32)
    buf_ref[...] = logits

    @pl.when(j == _NB)
    def _fin():
        g2 = g_ref[...]
        odd = (t_ref[...] & 1) == 1
        grow = jnp.where(odd, g2[:, _NF:], g2[:, :_NF])
        picked = jnp.sum(x * grow, axis=1, keepdims=True)
        logz = c_ref[...] + jnp.log2(s_ref[...])
        out_ref[...] = jnp.sum(logz - picked, axis=(0, 1), keepdims=True) * (
            _LN2 / _B)


def kernel(inputs, targets, features):
    x = inputs * (_INV_TEMP * _LOG2E)
    c = (jnp.sqrt(jnp.sum(x * x, axis=1, keepdims=True)) - _SHIFT)
    f2 = features.reshape(_ND // 2, 2 * _NF)
    g2 = _sc_gather(targets, f2)
    t = targets.reshape(_B, 1)
    out = pl.pallas_call(
        _loss_kernel,
        grid=(_NB + 1,),
        in_specs=[
            pl.BlockSpec((_B, _NF), lambda j: (0, 0)),
            pl.BlockSpec((_BN, _NF), lambda j: (jnp.minimum(j, _NB - 1), 0)),
            pl.BlockSpec((_B, 1), lambda j: (0, 0)),
            pl.BlockSpec((_B, 1), lambda j: (0, 0)),
            pl.BlockSpec((_B, 2 * _NF), lambda j: (0, 0)),
        ],
        out_specs=pl.BlockSpec((1, 1), lambda j: (0, 0)),
        out_shape=jax.ShapeDtypeStruct((1, 1), jnp.float32),
        scratch_shapes=[
            pltpu.VMEM((_B, 1), jnp.float32),
            pltpu.VMEM((_B, _BN), jnp.float32),
        ],
    )(x, features, c, t, g2)
    return out[0, 0]
